# baseline (speedup 1.0000x reference)
import functools

import jax
import jax.numpy as jnp
from jax import lax
from jax.experimental import pallas as pl
from jax.experimental.pallas import tpu as pltpu


def _tap_keeps(C_in, L, K, pad):
    lane = lax.broadcasted_iota(jnp.int32, (C_in, L), 1)
    keeps = {}
    for k in range(K):
        d = k - pad
        if d != 0:
            keeps[k] = (lane < L - d) if d > 0 else (lane >= -d)
    return keeps


def _sample_patches(xb, keeps, *, K, pad):
    rows = []
    for k in range(K):
        d = k - pad
        if d == 0:
            rows.append(xb)
        else:
            sh = pltpu.roll(xb, (-d) % xb.shape[1], 1)
            rows.append(jnp.where(keeps[k], sh, jnp.bfloat16(0)))
    return jnp.concatenate(rows, axis=0)


def _stats_body(x_hbm, w_ref, acc_ref, x_buf, in_sem, *, K, pad, steps, TS):
    sp = pl.program_id(0)
    base = sp * steps
    _, _, C_in, L = x_buf.shape
    C_out = w_ref.shape[0]

    def dma_in(slot, t):
        pltpu.make_async_copy(x_hbm.at[pl.ds((base + t) * TS, TS)],
                              x_buf.at[slot], in_sem.at[slot]).start()

    def wait_in(slot):
        pltpu.make_async_copy(x_hbm.at[pl.ds(0, TS)],
                              x_buf.at[slot], in_sem.at[slot]).wait()

    dma_in(0, 0)
    keeps = _tap_keeps(C_in, L, K, pad)
    w = w_ref[...]

    def body(t, carry):
        s, q = carry
        cur = lax.rem(t, 2)
        nxt = lax.rem(t + 1, 2)

        @pl.when(t + 1 < steps)
        def _():
            dma_in(nxt, t + 1)

        wait_in(cur)
        for n in range(TS):
            pm = _sample_patches(x_buf[cur, n].astype(jnp.bfloat16), keeps,
                                 K=K, pad=pad)
            c = jnp.dot(w, pm, preferred_element_type=jnp.float32)
            for j in range(0, L, 128):
                ch = c[:, j:j + 128]
                s = s + ch
                q = q + ch * ch
        return s, q

    zeros = jnp.zeros((C_out, 128), jnp.float32)
    s, q = lax.fori_loop(0, steps, body, (zeros, zeros))
    acc_ref[0] = jnp.concatenate([s, q], axis=1)


def _apply_body(x_hbm, w_ref, acc_ref, gb_ref, o_hbm, x_buf, o_buf,
                in_sem, out_sem, *, K, pad, steps, TN, count, eps):
    sp = pl.program_id(0)
    base = sp * steps
    _, _, C_in, L = x_buf.shape

    tot = jnp.sum(acc_ref[...], axis=0)
    s = jnp.sum(tot[:, 0:128], axis=1, keepdims=True)
    q = jnp.sum(tot[:, 128:256], axis=1, keepdims=True)
    mean = s / count
    var = jnp.maximum(q / count - mean * mean, 0.0)
    scale = gb_ref[:, 0:1] * lax.rsqrt(var + eps)
    shift = gb_ref[:, 1:2] - mean * scale
    w_bn = (w_ref[...].astype(jnp.float32) * scale).astype(jnp.bfloat16)

    def dma_in(slot, t):
        pltpu.make_async_copy(x_hbm.at[pl.ds((base + t) * TN, TN)],
                              x_buf.at[slot], in_sem.at[slot]).start()

    def wait_in(slot):
        pltpu.make_async_copy(x_hbm.at[pl.ds(0, TN)],
                              x_buf.at[slot], in_sem.at[slot]).wait()

    def dma_out(slot, t):
        pltpu.make_async_copy(o_buf.at[slot],
                              o_hbm.at[pl.ds((base + t) * TN, TN)],
                              out_sem.at[slot]).start()

    def wait_out(slot):
        pltpu.make_async_copy(o_buf.at[slot], o_hbm.at[pl.ds(0, TN)],
                              out_sem.at[slot]).wait()

    dma_in(0, 0)
    keeps = _tap_keeps(C_in, L, K, pad)

    def body(t, _):
        cur = lax.rem(t, 2)
        nxt = lax.rem(t + 1, 2)

        @pl.when(t + 1 < steps)
        def _():
            dma_in(nxt, t + 1)

        wait_in(cur)

        @pl.when(t >= 2)
        def _():
            wait_out(cur)

        for n in range(TN):
            pm = _sample_patches(x_buf[cur, n].astype(jnp.bfloat16), keeps,
                                 K=K, pad=pad)
            conv = jnp.dot(w_bn, pm, preferred_element_type=jnp.float32)
            o_buf[cur, n] = jnp.maximum(conv + shift, 0.0).astype(o_buf.dtype)
        dma_out(cur, t)
        return ()

    lax.fori_loop(0, steps, body, ())
    if steps >= 2:
        wait_out((steps - 2) % 2)
    wait_out((steps - 1) % 2)


def kernel(x, weight, bias, gamma, beta):
    del bias
    eps = 1e-5
    pad = 1
    N, C_in, L = x.shape
    C_out, _, K = weight.shape
    KC = K * C_in
    assert L + 2 * pad - K + 1 == L, "K=3, pad=1 keeps length"
    assert L % 128 == 0

    wf16 = jnp.transpose(weight, (0, 2, 1)).reshape(C_out, KC).astype(jnp.bfloat16)
    gb = jnp.stack([gamma, beta], axis=1).astype(jnp.float32)

    vmem = 52 * 1024 * 1024
    TS = 16
    while N % TS:
        TS -= 1
    s_tiles = N // TS
    ncore = 2 if (s_tiles % 2 == 0 and s_tiles >= 2) else 1
    steps1 = s_tiles // ncore

    acc = pl.pallas_call(
        functools.partial(_stats_body, K=K, pad=pad, steps=steps1, TS=TS),
        out_shape=jax.ShapeDtypeStruct((ncore, C_out, 256), jnp.float32),
        grid=(ncore,),
        in_specs=[
            pl.BlockSpec(memory_space=pl.ANY),
            pl.BlockSpec((C_out, KC), lambda sp: (0, 0)),
        ],
        out_specs=pl.BlockSpec((1, C_out, 256), lambda sp: (sp, 0, 0)),
        scratch_shapes=[
            pltpu.VMEM((2, TS, C_in, L), x.dtype),
            pltpu.SemaphoreType.DMA((2,)),
        ],
        compiler_params=pltpu.CompilerParams(
            dimension_semantics=("parallel",),
            vmem_limit_bytes=vmem),
    )(x, wf16)

    TN = 16
    while N % TN:
        TN -= 1
    n_tiles = N // TN
    steps2 = n_tiles // ncore

    out = pl.pallas_call(
        functools.partial(_apply_body, K=K, pad=pad, steps=steps2, TN=TN,
                          count=float(N * L), eps=eps),
        out_shape=jax.ShapeDtypeStruct((N, C_out, L), x.dtype),
        grid=(ncore,),
        in_specs=[
            pl.BlockSpec(memory_space=pl.ANY),
            pl.BlockSpec((C_out, KC), lambda sp: (0, 0)),
            pl.BlockSpec((ncore, C_out, 256), lambda sp: (0, 0, 0)),
            pl.BlockSpec((C_out, 2), lambda sp: (0, 0)),
        ],
        out_specs=pl.BlockSpec(memory_space=pl.ANY),
        scratch_shapes=[
            pltpu.VMEM((2, TN, C_in, L), x.dtype),
            pltpu.VMEM((2, TN, C_out, L), x.dtype),
            pltpu.SemaphoreType.DMA((2,)),
            pltpu.SemaphoreType.DMA((2,)),
        ],
        compiler_params=pltpu.CompilerParams(
            dimension_semantics=("parallel",),
            vmem_limit_bytes=vmem),
    )(x, wf16, acc, gb)
    return out
